# baseline (speedup 1.0000x reference)
import jax
import jax.numpy as jnp
import numpy as np
from jax import lax
from jax.experimental import pallas as pl
from jax.experimental.pallas import tpu as pltpu
from jax.sharding import Mesh, NamedSharding, PartitionSpec as P

_N = 8192
_FEAT = 128
_HID = 64
_EMB = 32

_BM = 256
_DM = 256


def _proj1_body(h_ref, w0_ref, w1_ref, hw0_ref, hw1_ref):
    h = h_ref[...]
    dims = (((1,), (1,)), ((), ()))
    hw0_ref[...] = lax.dot_general(h, w0_ref[...], dims,
                                   preferred_element_type=jnp.float32)
    hw1_ref[...] = lax.dot_general(h, w1_ref[...], dims,
                                   preferred_element_type=jnp.float32)


def _pass1_body(a0_ref, a1_ref, hw0_ref, hw1_ref, b1_ref, w20_ref, w21_ref,
                g0_ref, g1_ref):
    mm = (((1,), (0,)), ((), ()))
    acc = lax.dot_general(a0_ref[...], hw0_ref[...], mm,
                          preferred_element_type=jnp.float32)
    acc = acc + lax.dot_general(a1_ref[...], hw1_ref[...], mm,
                                preferred_element_type=jnp.float32)
    h1 = jnp.maximum(acc + b1_ref[...], 0.0)
    dims = (((1,), (1,)), ((), ()))
    g0_ref[...] = lax.dot_general(h1, w20_ref[...], dims,
                                  preferred_element_type=jnp.float32)
    g1_ref[...] = lax.dot_general(h1, w21_ref[...], dims,
                                  preferred_element_type=jnp.float32)


def _pass2_body(a0_ref, a1_ref, g0_ref, g1_ref, b2_ref, z_ref):
    mm = (((1,), (0,)), ((), ()))
    acc = lax.dot_general(a0_ref[...], g0_ref[...], mm,
                          preferred_element_type=jnp.float32)
    acc = acc + lax.dot_general(a1_ref[...], g1_ref[...], mm,
                                preferred_element_type=jnp.float32)
    z_ref[...] = acc + b2_ref[...]


def _decode_body(zi_ref, zj_ref, out_ref):
    dims = (((1,), (1,)), ((), ()))
    out_ref[...] = lax.dot_general(zi_ref[...], zj_ref[...], dims,
                                   preferred_element_type=jnp.float32)


def _full(shape):
    return pl.BlockSpec(shape, lambda i: (0, 0))


def _local_compute(H, a0, a1, W1_r0, W1_r1, b1_2d, W2_r0, W2_r1, b2_2d,
                   axis_name):
    rows = a0.shape[0]

    hw0, hw1 = pl.pallas_call(
        _proj1_body,
        grid=(8,),
        in_specs=[
            pl.BlockSpec((_N // 8, _FEAT), lambda i: (i, 0)),
            _full((_HID, _FEAT)),
            _full((_HID, _FEAT)),
        ],
        out_specs=[
            pl.BlockSpec((_N // 8, _HID), lambda i: (i, 0)),
            pl.BlockSpec((_N // 8, _HID), lambda i: (i, 0)),
        ],
        out_shape=[jax.ShapeDtypeStruct((_N, _HID), jnp.float32)] * 2,
        compiler_params=pltpu.CompilerParams(
            dimension_semantics=("parallel",)),
    )(H, W1_r0, W1_r1)

    g0_loc, g1_loc = pl.pallas_call(
        _pass1_body,
        grid=(rows // _BM,),
        in_specs=[
            pl.BlockSpec((_BM, _N), lambda i: (i, 0)),
            pl.BlockSpec((_BM, _N), lambda i: (i, 0)),
            _full((_N, _HID)),
            _full((_N, _HID)),
            _full((1, _HID)),
            _full((_EMB, _HID)),
            _full((_EMB, _HID)),
        ],
        out_specs=[
            pl.BlockSpec((_BM, _EMB), lambda i: (i, 0)),
            pl.BlockSpec((_BM, _EMB), lambda i: (i, 0)),
        ],
        out_shape=[jax.ShapeDtypeStruct((rows, _EMB), jnp.float32)] * 2,
        compiler_params=pltpu.CompilerParams(
            dimension_semantics=("parallel",)),
    )(a0, a1, hw0, hw1, b1_2d, W2_r0, W2_r1)

    if axis_name is not None:
        g0 = lax.all_gather(g0_loc, axis_name, axis=0, tiled=True)
        g1 = lax.all_gather(g1_loc, axis_name, axis=0, tiled=True)
    else:
        g0, g1 = g0_loc, g1_loc

    z_loc = pl.pallas_call(
        _pass2_body,
        grid=(rows // _BM,),
        in_specs=[
            pl.BlockSpec((_BM, _N), lambda i: (i, 0)),
            pl.BlockSpec((_BM, _N), lambda i: (i, 0)),
            _full((_N, _EMB)),
            _full((_N, _EMB)),
            _full((1, _EMB)),
        ],
        out_specs=pl.BlockSpec((_BM, _EMB), lambda i: (i, 0)),
        out_shape=jax.ShapeDtypeStruct((rows, _EMB), jnp.float32),
        compiler_params=pltpu.CompilerParams(
            dimension_semantics=("parallel",)),
    )(a0, a1, g0, g1, b2_2d)

    if axis_name is not None:
        z = lax.all_gather(z_loc, axis_name, axis=0, tiled=True)
    else:
        z = z_loc

    a_hat_loc = pl.pallas_call(
        _decode_body,
        grid=(rows // _DM,),
        in_specs=[
            pl.BlockSpec((_DM, _EMB), lambda i: (i, 0)),
            _full((_N, _EMB)),
        ],
        out_specs=pl.BlockSpec((_DM, _N), lambda i: (i, 0)),
        out_shape=jax.ShapeDtypeStruct((rows, _N), jnp.float32),
        compiler_params=pltpu.CompilerParams(
            dimension_semantics=("parallel",)),
    )(z_loc, z)

    return z_loc, a_hat_loc


def kernel(H, A_norm_r0, A_norm_r1, W1_r0, W1_r1, b1, W2_r0, W2_r1, b2):
    b1_2d = b1.reshape(1, _HID)
    b2_2d = b2.reshape(1, _EMB)

    devs = jax.devices()
    ndev = len(devs)
    while ndev > 1 and (_N // _BM) % ndev != 0:
        ndev -= 1

    if ndev == 1:
        z, a_hat = _local_compute(H, A_norm_r0, A_norm_r1, W1_r0, W1_r1,
                                  b1_2d, W2_r0, W2_r1, b2_2d, None)
        return (z, a_hat)

    mesh = Mesh(np.array(devs[:ndev]), ("x",))
    rep = P()
    rows = P("x", None)

    def _spmd(H, a0, a1, W1_r0, W1_r1, b1_2d, W2_r0, W2_r1, b2_2d):
        return _local_compute(H, a0, a1, W1_r0, W1_r1, b1_2d, W2_r0, W2_r1,
                              b2_2d, "x")

    z, a_hat = jax.shard_map(
        _spmd,
        mesh=mesh,
        in_specs=(rep, rows, rows, rep, rep, rep, rep, rep, rep),
        out_specs=(rows, rows),
        check_vma=False,
    )(H, A_norm_r0, A_norm_r1, W1_r0, W1_r1, b1_2d, W2_r0, W2_r1, b2_2d)

    return (z, a_hat)


# speedup vs baseline: 1.0677x; 1.0677x over previous
import jax
import jax.numpy as jnp
from jax import lax
from jax.experimental import pallas as pl
from jax.experimental.pallas import tpu as pltpu

_N = 8192
_FEAT = 128
_HID = 64
_EMB = 32

_BM = 256
_DM = 64

_MM = (((1,), (0,)), ((), ()))
_MT = (((1,), (1,)), ((), ()))


def _dot(a, b, dims):
    return lax.dot_general(a, b, dims, preferred_element_type=jnp.float32)


def _mega_body(a0_ref, a1_ref, h_ref, w10_ref, w11_ref, b1_ref,
               w20_ref, w21_ref, b2_ref,
               z_ref, ahat_ref,
               hw0_s, hw1_s, g0_s, g1_s, z_s):
    i = pl.program_id(0)
    p1 = _N // _BM
    p2 = 2 * p1

    @pl.when(i == 0)
    def _():
        h = h_ref[...]
        hw0_s[...] = _dot(h, w10_ref[...], _MT)
        hw1_s[...] = _dot(h, w11_ref[...], _MT)

    @pl.when(i < p1)
    def _():
        acc = _dot(a0_ref[...], hw0_s[...], _MM)
        acc = acc + _dot(a1_ref[...], hw1_s[...], _MM)
        h1 = jnp.maximum(acc + b1_ref[...], 0.0)
        g0_s[pl.ds(i * _BM, _BM), :] = _dot(h1, w20_ref[...], _MT)
        g1_s[pl.ds(i * _BM, _BM), :] = _dot(h1, w21_ref[...], _MT)

    @pl.when(jnp.logical_and(i >= p1, i < p2))
    def _():
        acc = _dot(a0_ref[...], g0_s[...], _MM)
        acc = acc + _dot(a1_ref[...], g1_s[...], _MM)
        zblk = acc + b2_ref[...]
        z_ref[...] = zblk
        z_s[pl.ds((i - p1) * _BM, _BM), :] = zblk

    @pl.when(i >= p2)
    def _():
        zi = z_s[pl.ds((i - p2) * _DM, _DM), :]
        ahat_ref[...] = _dot(zi, z_s[...], _MT)


def kernel(H, A_norm_r0, A_norm_r1, W1_r0, W1_r1, b1, W2_r0, W2_r1, b2):
    b1_2d = b1.reshape(1, _HID)
    b2_2d = b2.reshape(1, _EMB)

    p1 = _N // _BM
    p2 = 2 * p1
    p3 = p2 + _N // _DM

    def _a_idx(i):
        return (jnp.where(i < p1, i, jnp.where(i < p2, i - p1, p1 - 1)), 0)

    full = lambda shape: pl.BlockSpec(shape, lambda i: (0, 0))

    z, a_hat = pl.pallas_call(
        _mega_body,
        grid=(p3,),
        in_specs=[
            pl.BlockSpec((_BM, _N), _a_idx),
            pl.BlockSpec((_BM, _N), _a_idx),
            full((_N, _FEAT)),
            full((_HID, _FEAT)),
            full((_HID, _FEAT)),
            full((1, _HID)),
            full((_EMB, _HID)),
            full((_EMB, _HID)),
            full((1, _EMB)),
        ],
        out_specs=[
            pl.BlockSpec((_BM, _EMB), lambda i: (jnp.clip(i - p1, 0, p1 - 1), 0)),
            pl.BlockSpec((_DM, _N), lambda i: (jnp.maximum(i - p2, 0), 0)),
        ],
        out_shape=[
            jax.ShapeDtypeStruct((_N, _EMB), jnp.float32),
            jax.ShapeDtypeStruct((_N, _N), jnp.float32),
        ],
        scratch_shapes=[
            pltpu.VMEM((_N, _HID), jnp.float32),
            pltpu.VMEM((_N, _HID), jnp.float32),
            pltpu.VMEM((_N, _EMB), jnp.float32),
            pltpu.VMEM((_N, _EMB), jnp.float32),
            pltpu.VMEM((_N, _EMB), jnp.float32),
        ],
        compiler_params=pltpu.CompilerParams(
            dimension_semantics=("arbitrary",),
            vmem_limit_bytes=63 * 1024 * 1024),
    )(A_norm_r0, A_norm_r1, H, W1_r0, W1_r1, b1_2d, W2_r0, W2_r1, b2_2d)

    return (z, a_hat)


# speedup vs baseline: 1.1840x; 1.1089x over previous
import jax
import jax.numpy as jnp
from jax import lax
from jax.experimental import pallas as pl
from jax.experimental.pallas import tpu as pltpu

_N = 8192
_FEAT = 128
_HID = 64
_EMB = 32

_BM = 256
_DM = 512

_MM = (((1,), (0,)), ((), ()))
_MT = (((1,), (1,)), ((), ()))


def _dot(a, b, dims):
    return lax.dot_general(a, b, dims, preferred_element_type=jnp.float32)


def _pass1_body(a0_ref, a1_ref, h_ref, w10_ref, w11_ref, b1_ref,
                w20_ref, w21_ref, g0_ref, g1_ref, hw0_s, hw1_s):
    @pl.when(pl.program_id(0) == 0)
    def _():
        h = h_ref[...]
        hw0_s[...] = _dot(h, w10_ref[...], _MT)
        hw1_s[...] = _dot(h, w11_ref[...], _MT)

    acc = _dot(a0_ref[...], hw0_s[...], _MM)
    acc = acc + _dot(a1_ref[...], hw1_s[...], _MM)
    h1 = jnp.maximum(acc + b1_ref[...], 0.0)
    g0_ref[...] = _dot(h1, w20_ref[...], _MT)
    g1_ref[...] = _dot(h1, w21_ref[...], _MT)


def _pass2_body(a0_ref, a1_ref, g0_ref, g1_ref, b2_ref, z_ref):
    acc = _dot(a0_ref[...], g0_ref[...], _MM)
    acc = acc + _dot(a1_ref[...], g1_ref[...], _MM)
    z_ref[...] = acc + b2_ref[...]


def _decode_body(zi_ref, zj_ref, out_ref):
    out_ref[...] = _dot(zi_ref[...], zj_ref[...], _MT)


def _full(shape):
    return pl.BlockSpec(shape, lambda i: (0, 0))


def kernel(H, A_norm_r0, A_norm_r1, W1_r0, W1_r1, b1, W2_r0, W2_r1, b2):
    b1_2d = b1.reshape(1, _HID)
    b2_2d = b2.reshape(1, _EMB)

    g0, g1 = pl.pallas_call(
        _pass1_body,
        grid=(_N // _BM,),
        in_specs=[
            pl.BlockSpec((_BM, _N), lambda i: (i, 0)),
            pl.BlockSpec((_BM, _N), lambda i: (i, 0)),
            _full((_N, _FEAT)),
            _full((_HID, _FEAT)),
            _full((_HID, _FEAT)),
            _full((1, _HID)),
            _full((_EMB, _HID)),
            _full((_EMB, _HID)),
        ],
        out_specs=[
            pl.BlockSpec((_BM, _EMB), lambda i: (i, 0)),
            pl.BlockSpec((_BM, _EMB), lambda i: (i, 0)),
        ],
        out_shape=[jax.ShapeDtypeStruct((_N, _EMB), jnp.float32)] * 2,
        scratch_shapes=[
            pltpu.VMEM((_N, _HID), jnp.float32),
            pltpu.VMEM((_N, _HID), jnp.float32),
        ],
        compiler_params=pltpu.CompilerParams(
            dimension_semantics=("arbitrary",)),
    )(A_norm_r0, A_norm_r1, H, W1_r0, W1_r1, b1_2d, W2_r0, W2_r1)

    z = pl.pallas_call(
        _pass2_body,
        grid=(_N // _BM,),
        in_specs=[
            pl.BlockSpec((_BM, _N), lambda i: (i, 0)),
            pl.BlockSpec((_BM, _N), lambda i: (i, 0)),
            _full((_N, _EMB)),
            _full((_N, _EMB)),
            _full((1, _EMB)),
        ],
        out_specs=pl.BlockSpec((_BM, _EMB), lambda i: (i, 0)),
        out_shape=jax.ShapeDtypeStruct((_N, _EMB), jnp.float32),
        compiler_params=pltpu.CompilerParams(
            dimension_semantics=("parallel",)),
    )(A_norm_r0, A_norm_r1, g0, g1, b2_2d)

    a_hat = pl.pallas_call(
        _decode_body,
        grid=(_N // _DM,),
        in_specs=[
            pl.BlockSpec((_DM, _EMB), lambda i: (i, 0)),
            _full((_N, _EMB)),
        ],
        out_specs=pl.BlockSpec((_DM, _N), lambda i: (i, 0)),
        out_shape=jax.ShapeDtypeStruct((_N, _N), jnp.float32),
        compiler_params=pltpu.CompilerParams(
            dimension_semantics=("parallel",)),
    )(z, z)

    return (z, a_hat)


# speedup vs baseline: 1.1975x; 1.0114x over previous
import jax
import jax.numpy as jnp
from jax import lax
from jax.experimental import pallas as pl
from jax.experimental.pallas import tpu as pltpu

_N = 8192
_FEAT = 128
_HID = 64
_EMB = 32

_BM = 256
_DM = 256

_MM = (((1,), (0,)), ((), ()))
_MT = (((1,), (1,)), ((), ()))


def _dot(a, b, dims):
    return lax.dot_general(a, b, dims, preferred_element_type=jnp.float32)


def _pass1_body(a0_ref, a1_ref, h_ref, w10_ref, w11_ref, b1_ref,
                w20_ref, w21_ref, g0_ref, g1_ref, hw0_s, hw1_s):
    @pl.when(pl.program_id(0) == 0)
    def _():
        h = h_ref[...]
        hw0_s[...] = _dot(h, w10_ref[...], _MT)
        hw1_s[...] = _dot(h, w11_ref[...], _MT)

    acc = _dot(a0_ref[...], hw0_s[...], _MM)
    acc = acc + _dot(a1_ref[...], hw1_s[...], _MM)
    h1 = jnp.maximum(acc + b1_ref[...], 0.0)
    g0_ref[...] = _dot(h1, w20_ref[...], _MT)
    g1_ref[...] = _dot(h1, w21_ref[...], _MT)


def _pass2_decode_body(a0_ref, a1_ref, g0_ref, g1_ref, b2_ref,
                       z_ref, ahat_ref, z_s):
    i = pl.program_id(0)
    p1 = _N // _BM

    @pl.when(i < p1)
    def _():
        acc = _dot(a0_ref[...], g0_ref[...], _MM)
        acc = acc + _dot(a1_ref[...], g1_ref[...], _MM)
        zblk = acc + b2_ref[...]
        z_ref[...] = zblk
        z_s[pl.ds(i * _BM, _BM), :] = zblk

    @pl.when(i >= p1)
    def _():
        zi = z_s[pl.ds((i - p1) * _DM, _DM), :]
        ahat_ref[...] = _dot(zi, z_s[...], _MT)


def _full(shape):
    return pl.BlockSpec(shape, lambda i: (0, 0))


def kernel(H, A_norm_r0, A_norm_r1, W1_r0, W1_r1, b1, W2_r0, W2_r1, b2):
    b1_2d = b1.reshape(1, _HID)
    b2_2d = b2.reshape(1, _EMB)

    g0, g1 = pl.pallas_call(
        _pass1_body,
        grid=(_N // _BM,),
        in_specs=[
            pl.BlockSpec((_BM, _N), lambda i: (i, 0)),
            pl.BlockSpec((_BM, _N), lambda i: (i, 0)),
            _full((_N, _FEAT)),
            _full((_HID, _FEAT)),
            _full((_HID, _FEAT)),
            _full((1, _HID)),
            _full((_EMB, _HID)),
            _full((_EMB, _HID)),
        ],
        out_specs=[
            pl.BlockSpec((_BM, _EMB), lambda i: (i, 0)),
            pl.BlockSpec((_BM, _EMB), lambda i: (i, 0)),
        ],
        out_shape=[jax.ShapeDtypeStruct((_N, _EMB), jnp.float32)] * 2,
        scratch_shapes=[
            pltpu.VMEM((_N, _HID), jnp.float32),
            pltpu.VMEM((_N, _HID), jnp.float32),
        ],
        compiler_params=pltpu.CompilerParams(
            dimension_semantics=("arbitrary",),
            vmem_limit_bytes=63 * 1024 * 1024),
    )(A_norm_r0, A_norm_r1, H, W1_r0, W1_r1, b1_2d, W2_r0, W2_r1)

    p1 = _N // _BM
    grid2 = p1 + _N // _DM

    def _a_idx(i):
        return (jnp.minimum(i, p1 - 1), 0)

    z, a_hat = pl.pallas_call(
        _pass2_decode_body,
        grid=(grid2,),
        in_specs=[
            pl.BlockSpec((_BM, _N), _a_idx),
            pl.BlockSpec((_BM, _N), _a_idx),
            _full((_N, _EMB)),
            _full((_N, _EMB)),
            _full((1, _EMB)),
        ],
        out_specs=[
            pl.BlockSpec((_BM, _EMB), lambda i: (jnp.minimum(i, p1 - 1), 0)),
            pl.BlockSpec((_DM, _N), lambda i: (jnp.maximum(i - p1, 0), 0)),
        ],
        out_shape=[
            jax.ShapeDtypeStruct((_N, _EMB), jnp.float32),
            jax.ShapeDtypeStruct((_N, _N), jnp.float32),
        ],
        scratch_shapes=[pltpu.VMEM((_N, _EMB), jnp.float32)],
        compiler_params=pltpu.CompilerParams(
            dimension_semantics=("arbitrary",),
            vmem_limit_bytes=63 * 1024 * 1024),
    )(A_norm_r0, A_norm_r1, g0, g1, b2_2d)

    return (z, a_hat)
